# baseline (speedup 1.0000x reference)
import functools

import jax
import jax.numpy as jnp
from jax import lax
from jax.experimental import pallas as pl
from jax.experimental.pallas import tpu as pltpu
from jax.experimental.pallas import tpu_sc as plsc

_K = 128
_L = 16
_LB = 32
_NJ = _K // _L
_NJB = _K // _LB
_ONE_HOP = 32
_TWO_HOP = 64
_DROWS = 1 + _TWO_HOP
_NC = 2
_NS = 16
_NW = _NC * _NS
_NBUF = 2


def _permute(x, idx):
    return lax.gather(
        x, idx[:, None],
        lax.GatherDimensionNumbers(
            offset_dims=(), collapsed_slice_dims=(0,), start_index_map=(0,)),
        (1,), mode=lax.GatherScatterMode.PROMISE_IN_BOUNDS)


def _combine_row(dr, wr, g, out_v):
    inv1 = jnp.float32(1.0 / _ONE_HOP)
    inv2 = jnp.float32(1.0 / _TWO_HOP)

    def widen(w):
        fe = lax.bitcast_convert_type(w << 16, jnp.float32)
        fo = lax.bitcast_convert_type(w, jnp.float32)
        return fe, fo

    def load(ref, r):
        out = []
        for j in range(_NJB):
            fe, fo = widen(ref[r, pl.ds(j * _L, _L)])
            out += [fe, fo]
        return out

    def acc_doc(r, acc):
        return [a + b for a, b in zip(acc, load(dr, r))]

    def acc_word(r, acc):
        return [a + b for a, b in zip(acc, load(wr, r))]

    two = lax.fori_loop(2, _DROWS, acc_doc, load(dr, 1), unroll=8)
    one = lax.fori_loop(1, _ONE_HOP, acc_word, load(wr, 0), unroll=8)
    x = load(dr, 0)
    t = [x[j] + two[j] * inv2 + one[j] * inv1 for j in range(_NJ)]

    m16 = t[0]
    for j in range(1, _NJ):
        m16 = jnp.maximum(m16, t[j])
    lanes = lax.iota(jnp.int32, _L)
    for k in (8, 4, 2, 1):
        m16 = jnp.maximum(m16, _permute(m16, lanes ^ k))
    e = [jnp.exp(t[j] - m16) for j in range(_NJ)]
    s16 = e[0]
    for j in range(1, _NJ):
        s16 = s16 + e[j]
    for k in (8, 4, 2, 1):
        s16 = s16 + _permute(s16, lanes ^ k)
    r = 1.0 / s16
    half = lanes >> 1
    parity = (lanes & 1) == 1
    for j in range(_NJB):
        fe = e[2 * j] * r
        fo = e[2 * j + 1] * r
        lo = jnp.where(parity, _permute(fo, half), _permute(fe, half))
        hi = jnp.where(parity, _permute(fo, 8 + half), _permute(fe, 8 + half))
        out_v[g, pl.ds(j * _LB, _L)] = lo
        out_v[g, pl.ds(j * _LB + _L, _L)] = hi


def kernel(v, one_hop_list, two_hop_list, doc_topic_dist, word_topic_dist):
    B = v.shape[0]
    assert B % (_NW * _NBUF) == 0
    ipw = B // _NW

    def _pack_table(tab):
        bf = tab.astype(jnp.bfloat16)
        return lax.bitcast_convert_type(
            bf.reshape(tab.shape[0], _K // 2, 2), jnp.int32)

    doc_bf = _pack_table(doc_topic_dist)
    word_bf = _pack_table(word_topic_dist)
    doc_idx = jnp.concatenate(
        [v.astype(jnp.int32)[:, None], two_hop_list.astype(jnp.int32)], axis=1)
    word_idx = one_hop_list.astype(jnp.int32)

    mesh = plsc.VectorSubcoreMesh(
        core_axis_name="c", subcore_axis_name="s",
        num_cores=_NC, num_subcores=_NS)

    @functools.partial(
        pl.kernel,
        out_type=jax.ShapeDtypeStruct((B, _K), jnp.float32),
        mesh=mesh,
        compiler_params=pltpu.CompilerParams(use_tc_tiling_on_sc=False),
        scratch_types=[
            pltpu.VMEM((ipw, _DROWS), jnp.int32),
            pltpu.VMEM((ipw, _ONE_HOP), jnp.int32),
            pltpu.VMEM((_NBUF, _DROWS, _K // 2), jnp.int32),
            pltpu.VMEM((_NBUF, _ONE_HOP, _K // 2), jnp.int32),
            pltpu.VMEM((ipw, _K), jnp.float32),
            [pltpu.SemaphoreType.DMA] * _NBUF,
            [pltpu.SemaphoreType.DMA] * _NBUF,
        ],
    )
    def run(doc_tab, word_tab, didx_hbm, widx_hbm, out_hbm,
            didx_v, widx_v, drows, wrows, out_v, dsems, wsems):
        wid = lax.axis_index("s") * _NC + lax.axis_index("c")
        base = wid * ipw
        pltpu.sync_copy(didx_hbm.at[pl.ds(base, ipw)], didx_v)
        pltpu.sync_copy(widx_hbm.at[pl.ds(base, ipw)], widx_v)

        def issue(g, slot):
            pltpu.async_copy(doc_tab.at[didx_v.at[g]], drows.at[slot],
                             dsems[slot])
            pltpu.async_copy(word_tab.at[widx_v.at[g]], wrows.at[slot],
                             wsems[slot])

        def wait(g, slot):
            pltpu.make_async_copy(doc_tab.at[didx_v.at[g]], drows.at[slot],
                                  dsems[slot]).wait()
            pltpu.make_async_copy(word_tab.at[widx_v.at[g]], wrows.at[slot],
                                  wsems[slot]).wait()

        for b in range(_NBUF):
            issue(b, b)

        def group(p, carry):
            for b in range(_NBUF):
                g = p * _NBUF + b
                wait(g, b)
                _combine_row(drows.at[b], wrows.at[b], g, out_v)

                @pl.when(g + _NBUF < ipw)
                def _prefetch(b=b, g=g):
                    issue(g + _NBUF, b)
            return carry

        lax.fori_loop(0, ipw // _NBUF, group, 0)
        pltpu.sync_copy(out_v, out_hbm.at[pl.ds(base, ipw)])

    return run(doc_bf, word_bf, doc_idx, word_idx)


# speedup vs baseline: 3.5188x; 3.5188x over previous
import functools

import jax
import jax.numpy as jnp
from jax import lax
from jax.experimental import pallas as pl
from jax.experimental.pallas import tpu as pltpu
from jax.experimental.pallas import tpu_sc as plsc

_K = 128
_L = 16
_NJ = _K // _L
_ONE_HOP = 32
_TWO_HOP = 64
_NC = 2
_NS = 16
_NW = _NC * _NS
_NBUF = 2


def _permute(x, idx):
    return lax.gather(
        x, idx[:, None],
        lax.GatherDimensionNumbers(
            offset_dims=(), collapsed_slice_dims=(0,), start_index_map=(0,)),
        (1,), mode=lax.GatherScatterMode.PROMISE_IN_BOUNDS)


def _combine_row(xr, dr, wr, g, out_v):
    inv1 = 1.0 / _ONE_HOP
    inv2 = 1.0 / _TWO_HOP

    def acc_doc(r, acc):
        return [acc[j] + dr[r, pl.ds(j * _L, _L)] for j in range(_NJ)]

    def acc_word(r, acc):
        return [acc[j] + wr[r, pl.ds(j * _L, _L)] for j in range(_NJ)]

    two = lax.fori_loop(
        1, _TWO_HOP, acc_doc,
        [dr[0, pl.ds(j * _L, _L)] for j in range(_NJ)], unroll=8)
    one = lax.fori_loop(
        1, _ONE_HOP, acc_word,
        [wr[0, pl.ds(j * _L, _L)] for j in range(_NJ)], unroll=8)
    t = [xr[g, pl.ds(j * _L, _L)] + two[j] * inv2 + one[j] * inv1
         for j in range(_NJ)]

    m16 = t[0]
    for j in range(1, _NJ):
        m16 = jnp.maximum(m16, t[j])
    lanes = lax.iota(jnp.int32, _L)
    for k in (8, 4, 2, 1):
        m16 = jnp.maximum(m16, _permute(m16, lanes ^ k))
    e = [jnp.exp(t[j] - m16) for j in range(_NJ)]
    s16 = e[0]
    for j in range(1, _NJ):
        s16 = s16 + e[j]
    for k in (8, 4, 2, 1):
        s16 = s16 + _permute(s16, lanes ^ k)
    r = 1.0 / s16
    for j in range(_NJ):
        out_v[g, pl.ds(j * _L, _L)] = e[j] * r


def kernel(v, one_hop_list, two_hop_list, doc_topic_dist, word_topic_dist):
    B = v.shape[0]
    assert B % (_NW * _NBUF) == 0
    ipw = B // _NW

    v_idx = v.astype(jnp.int32)
    doc_idx = two_hop_list.astype(jnp.int32).reshape(-1)
    word_idx = one_hop_list.astype(jnp.int32).reshape(-1)

    mesh = plsc.VectorSubcoreMesh(
        core_axis_name="c", subcore_axis_name="s",
        num_cores=_NC, num_subcores=_NS)

    @functools.partial(
        pl.kernel,
        out_type=jax.ShapeDtypeStruct((B, _K), jnp.float32),
        mesh=mesh,
        scratch_types=[
            pltpu.VMEM((ipw,), jnp.int32),
            pltpu.VMEM((ipw * _TWO_HOP,), jnp.int32),
            pltpu.VMEM((ipw * _ONE_HOP,), jnp.int32),
            pltpu.VMEM((ipw, _K), jnp.float32),
            pltpu.VMEM((_NBUF, _TWO_HOP, _K), jnp.float32),
            pltpu.VMEM((_NBUF, _ONE_HOP, _K), jnp.float32),
            pltpu.VMEM((ipw, _K), jnp.float32),
            [pltpu.SemaphoreType.DMA] * _NBUF,
            [pltpu.SemaphoreType.DMA] * _NBUF,
            pltpu.SemaphoreType.DMA,
        ],
    )
    def run(doc_tab, word_tab, vidx_hbm, didx_hbm, widx_hbm, out_hbm,
            vidx_v, didx_v, widx_v, xrows, drows, wrows, out_v,
            dsems, wsems, xsem):
        wid = lax.axis_index("s") * _NC + lax.axis_index("c")
        base = wid * ipw
        pltpu.sync_copy(vidx_hbm.at[pl.ds(base, ipw)], vidx_v)
        pltpu.sync_copy(didx_hbm.at[pl.ds(base * _TWO_HOP, ipw * _TWO_HOP)],
                        didx_v)
        pltpu.sync_copy(widx_hbm.at[pl.ds(base * _ONE_HOP, ipw * _ONE_HOP)],
                        widx_v)
        nx = 128
        cx = [pltpu.async_copy(doc_tab.at[vidx_v.at[pl.ds(h * nx, nx)]],
                               xrows.at[pl.ds(h * nx, nx)], xsem)
              for h in range(ipw // nx)]
        for c in cx:
            c.wait()

        def issue(g, slot):
            pltpu.async_copy(
                doc_tab.at[didx_v.at[pl.ds(g * _TWO_HOP, _TWO_HOP)]],
                drows.at[slot], dsems[slot])
            pltpu.async_copy(
                word_tab.at[widx_v.at[pl.ds(g * _ONE_HOP, _ONE_HOP)]],
                wrows.at[slot], wsems[slot])

        def wait(g, slot):
            pltpu.make_async_copy(
                doc_tab.at[didx_v.at[pl.ds(g * _TWO_HOP, _TWO_HOP)]],
                drows.at[slot], dsems[slot]).wait()
            pltpu.make_async_copy(
                word_tab.at[widx_v.at[pl.ds(g * _ONE_HOP, _ONE_HOP)]],
                wrows.at[slot], wsems[slot]).wait()

        for b in range(_NBUF):
            issue(b, b)

        def group(p, carry):
            for b in range(_NBUF):
                g = p * _NBUF + b
                wait(g, b)
                _combine_row(xrows, drows.at[b], wrows.at[b], g, out_v)

                @pl.when(g + _NBUF < ipw)
                def _prefetch(b=b, g=g):
                    issue(g + _NBUF, b)
            return carry

        lax.fori_loop(0, ipw // _NBUF, group, 0)
        pltpu.sync_copy(out_v, out_hbm.at[pl.ds(base, ipw)])

    return run(doc_topic_dist, word_topic_dist, v_idx, doc_idx, word_idx)
